# Initial kernel scaffold; baseline (speedup 1.0000x reference)
import jax
import jax.numpy as jnp
from jax.experimental import pallas as pl


def kernel(h, edge_index, edge_weight, W1, b1, W2, b2, W3, b3, fcW1, fcb1, fcW2, fcb2, fcW3, fcb3):
    raise NotImplementedError("write your pallas kernel here")


# speedup vs baseline: 10.0109x; 10.0109x over previous
import functools

import jax
import jax.numpy as jnp
from jax import lax
from jax.experimental import pallas as pl
from jax.experimental.pallas import tpu as pltpu
from jax.experimental.pallas import tpu_sc as plsc

NC = 2
NS = 16
ROW = 128
D = 5


def _edge_pass(n, e, with_count):
    npl = 6 if with_count else 5
    r = e // ROW
    rb, rem = r // (NC * NS), r % (NC * NS)
    ch = (-(-n // NS) + 7) // 8 * 8
    last = n - (NS - 1) * ch

    mesh = plsc.VectorSubcoreMesh(
        core_axis_name="c", subcore_axis_name="s", num_cores=NC,
        num_subcores=NS)

    scratch = (
        [pltpu.VMEM_SHARED((n,), jnp.float32) for _ in range(D)]
        + [pltpu.VMEM_SHARED((n,), jnp.float32) for _ in range(npl)]
        + [pltpu.VMEM((ch,), jnp.float32),
           pltpu.VMEM((ROW,), jnp.int32),
           pltpu.VMEM((ROW,), jnp.int32),
           pltpu.VMEM((ROW,), jnp.float32)]
        + [pltpu.VMEM((ROW,), jnp.float32) for _ in range(D)]
        + [pltpu.VMEM((ROW,), jnp.float32)]
    )

    @functools.partial(
        pl.kernel,
        out_type=jax.ShapeDtypeStruct((NC * npl * n,), jnp.float32),
        mesh=mesh, scratch_types=scratch)
    def edge_pass(*a):
        h_in = a[0:D]
        src1, dst1, ew1, out = a[D], a[D + 1], a[D + 2], a[D + 3]
        sc = a[D + 4:]
        h_sh = sc[0:D]
        acc = sc[D:D + npl]
        zbuf, src_v, dst_v, ew_v = sc[D + npl:D + npl + 4]
        g = sc[D + npl + 4:D + npl + 4 + D]
        ones_v = sc[D + npl + 4 + D]

        cid = lax.axis_index("c")
        sid = lax.axis_index("s")
        off = sid * ch

        def ofill(i, _):
            ones_v[pl.ds(i * 16, 16)] = jnp.ones((16,), jnp.float32)
            return 0
        lax.fori_loop(0, ROW // 16, ofill, 0)

        @pl.when(sid < NS - 1)
        def _():
            for j in range(D):
                pltpu.sync_copy(h_in[j].at[pl.ds(off, ch)], zbuf)
                pltpu.sync_copy(zbuf, h_sh[j].at[pl.ds(off, ch)])

        @pl.when(sid == NS - 1)
        def _():
            for j in range(D):
                pltpu.sync_copy(h_in[j].at[pl.ds(off, last)],
                                zbuf.at[pl.ds(0, last)])
                pltpu.sync_copy(zbuf.at[pl.ds(0, last)],
                                h_sh[j].at[pl.ds(off, last)])

        def zfill(i, _):
            zbuf[pl.ds(i * 16, 16)] = jnp.zeros((16,), jnp.float32)
            return 0
        lax.fori_loop(0, ch // 16, zfill, 0)

        @pl.when(sid < NS - 1)
        def _():
            for j in range(npl):
                pltpu.sync_copy(zbuf, acc[j].at[pl.ds(off, ch)])

        @pl.when(sid == NS - 1)
        def _():
            for j in range(npl):
                pltpu.sync_copy(zbuf.at[pl.ds(0, last)],
                                acc[j].at[pl.ds(off, last)])

        plsc.subcore_barrier()

        wid = cid * NS + sid
        base = wid * rb + jnp.minimum(wid, rem)
        nrows = rb + jnp.where(wid < rem, 1, 0)

        def row_body(t, _):
            eoff = (base + t) * ROW
            pltpu.sync_copy(src1.at[pl.ds(eoff, ROW)], src_v)
            pltpu.sync_copy(ew1.at[pl.ds(eoff, ROW)], ew_v)
            pltpu.sync_copy(dst1.at[pl.ds(eoff, ROW)], dst_v)
            for j in range(D):
                pltpu.sync_copy(h_sh[j].at[src_v], g[j])
            for k in range(ROW // 16):
                sl = pl.ds(k * 16, 16)
                w = ew_v[sl]
                for j in range(D):
                    g[j][sl] = g[j][sl] * w
            for j in range(D):
                pltpu.sync_copy(g[j], acc[j].at[dst_v], add=True)
            if with_count:
                pltpu.sync_copy(ones_v, acc[D].at[dst_v], add=True)
            return 0
        lax.fori_loop(0, nrows, row_body, 0)

        plsc.subcore_barrier()

        @pl.when(sid < NS - 1)
        def _():
            for j in range(npl):
                pltpu.sync_copy(acc[j].at[pl.ds(off, ch)], zbuf)
                pltpu.sync_copy(zbuf,
                                out.at[pl.ds((cid * npl + j) * n + off, ch)])

        @pl.when(sid == NS - 1)
        def _():
            for j in range(npl):
                pltpu.sync_copy(acc[j].at[pl.ds(off, last)],
                                zbuf.at[pl.ds(0, last)])
                pltpu.sync_copy(zbuf.at[pl.ds(0, last)],
                                out.at[pl.ds((cid * npl + j) * n + off, last)])

    return edge_pass


def _dense_first(n):
    def body(part, wt, b, out, cnt_out):
        p = part[0] + part[1]
        c = jnp.maximum(p[D:D + 1, :], 1.0)
        m = p[0:D, :] / c
        x = jnp.dot(wt[...], m, preferred_element_type=jnp.float32) + b[...]
        out[...] = jnp.maximum(x, 0.0)
        cnt_out[...] = c

    return pl.pallas_call(
        body,
        out_shape=[jax.ShapeDtypeStruct((D, n), jnp.float32),
                   jax.ShapeDtypeStruct((1, n), jnp.float32)])


def _dense_mid(n):
    def body(part, cnt, wt, b, out):
        p = part[0] + part[1]
        m = p / cnt[...]
        x = jnp.dot(wt[...], m, preferred_element_type=jnp.float32) + b[...]
        out[...] = jnp.maximum(x, 0.0)

    return pl.pallas_call(
        body, out_shape=jax.ShapeDtypeStruct((D, n), jnp.float32))


def _dense_final(n):
    def body(part, cnt, wt3, b3, fwt1, fb1, fwt2, fb2, fwt3, fb3, out):
        p = part[0] + part[1]
        m = p / cnt[...]
        x = jnp.dot(wt3[...], m, preferred_element_type=jnp.float32) + b3[...]
        h = jnp.maximum(x, 0.0)
        x = jnp.dot(fwt1[...], h, preferred_element_type=jnp.float32) + fb1[...]
        h = jnp.maximum(x, 0.0)
        x = jnp.dot(fwt2[...], h, preferred_element_type=jnp.float32) + fb2[...]
        h = jnp.maximum(x, 0.0)
        out[...] = (jnp.dot(fwt3[...], h, preferred_element_type=jnp.float32)
                    + fb3[...])

    return pl.pallas_call(
        body, out_shape=jax.ShapeDtypeStruct((D, n), jnp.float32))


@functools.lru_cache(maxsize=4)
def _build(n, e):
    return (_edge_pass(n, e, True), _edge_pass(n, e, False),
            _dense_first(n), _dense_mid(n), _dense_final(n))


def kernel(h, edge_index, edge_weight, W1, b1, W2, b2, W3, b3,
           fcW1, fcb1, fcW2, fcb2, fcW3, fcb3):
    n, d = h.shape
    e = edge_weight.shape[0]
    assert d == D and e % ROW == 0
    ep1, ep, dfirst, dmid, dfinal = _build(n, e)

    src1 = edge_index[0]
    dst1 = edge_index[1]
    ew1 = edge_weight

    planes = [h[:, j] for j in range(D)]

    part = ep1(*planes, src1, dst1, ew1).reshape(NC, 6, n)
    h1, cnt = dfirst(part, W1.T, b1[:, None])

    part = ep(*[h1[j] for j in range(D)], src1, dst1, ew1).reshape(NC, D, n)
    h2 = dmid(part, cnt, W2.T, b2[:, None])

    part = ep(*[h2[j] for j in range(D)], src1, dst1, ew1).reshape(NC, D, n)
    y = dfinal(part, cnt, W3.T, b3[:, None], fcW1.T, fcb1[:, None],
               fcW2.T, fcb2[:, None], fcW3.T, fcb3[:, None])

    return y.T


# speedup vs baseline: 27.8961x; 2.7866x over previous
import functools

import jax
import jax.numpy as jnp
from jax import lax
from jax.experimental import pallas as pl
from jax.experimental.pallas import tpu as pltpu
from jax.experimental.pallas import tpu_sc as plsc

NC = 2
NS = 16
ROW = 128
D = 5


def _edge_pass(n, e, with_count):
    npl = 6 if with_count else 5
    CR = 4
    CE = CR * ROW
    nchunks = e // CE
    cb, crem = nchunks // (NC * NS), nchunks % (NC * NS)
    ch = (-(-n // NS) + 7) // 8 * 8
    last = n - (NS - 1) * ch

    mesh = plsc.VectorSubcoreMesh(
        core_axis_name="c", subcore_axis_name="s", num_cores=NC,
        num_subcores=NS)

    scratch = (
        [pltpu.VMEM_SHARED((n,), jnp.float32) for _ in range(D)]
        + [pltpu.VMEM_SHARED((n,), jnp.float32) for _ in range(npl)]
        + [pltpu.VMEM((ch,), jnp.float32),
           pltpu.VMEM((CR, ROW), jnp.int32),
           pltpu.VMEM((CR, ROW), jnp.int32),
           pltpu.VMEM((CR, ROW), jnp.float32)]
        + [pltpu.VMEM((CR, ROW), jnp.float32) for _ in range(D)]
        + [pltpu.VMEM((ROW,), jnp.float32),
           pltpu.SemaphoreType.DMA,
           pltpu.SemaphoreType.DMA]
    )

    @functools.partial(
        pl.kernel,
        out_type=jax.ShapeDtypeStruct((NC * npl * n,), jnp.float32),
        mesh=mesh, scratch_types=scratch)
    def edge_pass(*a):
        h_in = a[0:D]
        src1, dst1, ew1, out = a[D], a[D + 1], a[D + 2], a[D + 3]
        sc = a[D + 4:]
        h_sh = sc[0:D]
        acc = sc[D:D + npl]
        zbuf, src_v, dst_v, ew_v = sc[D + npl:D + npl + 4]
        g = sc[D + npl + 4:D + npl + 4 + D]
        ones_v, sem_g, sem_s = sc[D + npl + 4 + D:D + npl + 4 + D + 3]

        cid = lax.axis_index("c")
        sid = lax.axis_index("s")
        off = sid * ch

        def ofill(i, _):
            ones_v[pl.ds(i * 16, 16)] = jnp.ones((16,), jnp.float32)
            return 0
        lax.fori_loop(0, ROW // 16, ofill, 0)

        @pl.when(sid < NS - 1)
        def _():
            for j in range(D):
                pltpu.sync_copy(h_in[j].at[pl.ds(off, ch)], zbuf)
                pltpu.sync_copy(zbuf, h_sh[j].at[pl.ds(off, ch)])

        @pl.when(sid == NS - 1)
        def _():
            for j in range(D):
                pltpu.sync_copy(h_in[j].at[pl.ds(off, last)],
                                zbuf.at[pl.ds(0, last)])
                pltpu.sync_copy(zbuf.at[pl.ds(0, last)],
                                h_sh[j].at[pl.ds(off, last)])

        def zfill(i, _):
            zbuf[pl.ds(i * 16, 16)] = jnp.zeros((16,), jnp.float32)
            return 0
        lax.fori_loop(0, ch // 16, zfill, 0)

        @pl.when(sid < NS - 1)
        def _():
            for j in range(npl):
                pltpu.sync_copy(zbuf, acc[j].at[pl.ds(off, ch)])

        @pl.when(sid == NS - 1)
        def _():
            for j in range(npl):
                pltpu.sync_copy(zbuf.at[pl.ds(0, last)],
                                acc[j].at[pl.ds(off, last)])

        plsc.subcore_barrier()

        wid = cid * NS + sid
        base = wid * cb + jnp.minimum(wid, crem)
        nch = cb + jnp.where(wid < crem, 1, 0)

        def chunk_body(t, _):
            roff = (base + t) * CR
            pltpu.sync_copy(src1.at[pl.ds(roff, CR), :], src_v)
            pltpu.sync_copy(ew1.at[pl.ds(roff, CR), :], ew_v)
            pltpu.sync_copy(dst1.at[pl.ds(roff, CR), :], dst_v)
            gd = []
            for q in range(CR):
                for j in range(D):
                    gd.append(pltpu.async_copy(
                        h_sh[j].at[src_v.at[q]], g[j].at[q], sem_g))
            for dsc in gd:
                dsc.wait()
            for q in range(CR):
                for k in range(ROW // 16):
                    sl = pl.ds(k * 16, 16)
                    w = ew_v.at[q][sl]
                    for j in range(D):
                        g[j].at[q][sl] = g[j].at[q][sl] * w
            sd = []
            for q in range(CR):
                for j in range(D):
                    sd.append(pltpu.async_copy(
                        g[j].at[q], acc[j].at[dst_v.at[q]], sem_s, add=True))
                if with_count:
                    sd.append(pltpu.async_copy(
                        ones_v, acc[D].at[dst_v.at[q]], sem_s, add=True))
            for dsc in sd:
                dsc.wait()
            return 0
        lax.fori_loop(0, nch, chunk_body, 0)

        plsc.subcore_barrier()

        @pl.when(sid < NS - 1)
        def _():
            for j in range(npl):
                pltpu.sync_copy(acc[j].at[pl.ds(off, ch)], zbuf)
                pltpu.sync_copy(zbuf,
                                out.at[pl.ds((cid * npl + j) * n + off, ch)])

        @pl.when(sid == NS - 1)
        def _():
            for j in range(npl):
                pltpu.sync_copy(acc[j].at[pl.ds(off, last)],
                                zbuf.at[pl.ds(0, last)])
                pltpu.sync_copy(zbuf.at[pl.ds(0, last)],
                                out.at[pl.ds((cid * npl + j) * n + off, last)])

    return edge_pass


def _dense_first(n):
    def body(part, wt, b, out, cnt_out):
        p = part[0] + part[1]
        c = jnp.maximum(p[D:D + 1, :], 1.0)
        m = p[0:D, :] / c
        x = jnp.dot(wt[...], m, preferred_element_type=jnp.float32) + b[...]
        out[...] = jnp.maximum(x, 0.0)
        cnt_out[...] = c

    return pl.pallas_call(
        body,
        out_shape=[jax.ShapeDtypeStruct((D, n), jnp.float32),
                   jax.ShapeDtypeStruct((1, n), jnp.float32)])


def _dense_mid(n):
    def body(part, cnt, wt, b, out):
        p = part[0] + part[1]
        m = p / cnt[...]
        x = jnp.dot(wt[...], m, preferred_element_type=jnp.float32) + b[...]
        out[...] = jnp.maximum(x, 0.0)

    return pl.pallas_call(
        body, out_shape=jax.ShapeDtypeStruct((D, n), jnp.float32))


def _dense_final(n):
    def body(part, cnt, wt3, b3, fwt1, fb1, fwt2, fb2, fwt3, fb3, out):
        p = part[0] + part[1]
        m = p / cnt[...]
        x = jnp.dot(wt3[...], m, preferred_element_type=jnp.float32) + b3[...]
        h = jnp.maximum(x, 0.0)
        x = jnp.dot(fwt1[...], h, preferred_element_type=jnp.float32) + fb1[...]
        h = jnp.maximum(x, 0.0)
        x = jnp.dot(fwt2[...], h, preferred_element_type=jnp.float32) + fb2[...]
        h = jnp.maximum(x, 0.0)
        out[...] = (jnp.dot(fwt3[...], h, preferred_element_type=jnp.float32)
                    + fb3[...])

    return pl.pallas_call(
        body, out_shape=jax.ShapeDtypeStruct((D, n), jnp.float32))


@functools.lru_cache(maxsize=4)
def _build(n, e):
    return (_edge_pass(n, e, True), _edge_pass(n, e, False),
            _dense_first(n), _dense_mid(n), _dense_final(n))


def kernel(h, edge_index, edge_weight, W1, b1, W2, b2, W3, b3,
           fcW1, fcb1, fcW2, fcb2, fcW3, fcb3):
    n, d = h.shape
    e = edge_weight.shape[0]
    assert d == D and e % ROW == 0
    ep1, ep, dfirst, dmid, dfinal = _build(n, e)

    src1 = edge_index[0].reshape(e // ROW, ROW)
    dst1 = edge_index[1].reshape(e // ROW, ROW)
    ew1 = edge_weight.reshape(e // ROW, ROW)

    planes = [h[:, j] for j in range(D)]

    part = ep1(*planes, src1, dst1, ew1).reshape(NC, 6, n)
    h1, cnt = dfirst(part, W1.T, b1[:, None])

    part = ep(*[h1[j] for j in range(D)], src1, dst1, ew1).reshape(NC, D, n)
    h2 = dmid(part, cnt, W2.T, b2[:, None])

    part = ep(*[h2[j] for j in range(D)], src1, dst1, ew1).reshape(NC, D, n)
    y = dfinal(part, cnt, W3.T, b3[:, None], fcW1.T, fcb1[:, None],
               fcW2.T, fcb2[:, None], fcW3.T, fcb3[:, None])

    return y.T


# speedup vs baseline: 37.2178x; 1.3342x over previous
import functools

import jax
import jax.numpy as jnp
from jax import lax
from jax.experimental import pallas as pl
from jax.experimental.pallas import tpu as pltpu
from jax.experimental.pallas import tpu_sc as plsc

NC = 2
NS = 16
ROW = 128
D = 5


def _edge_pass(n, e, with_count):
    npl = 6 if with_count else 5
    CR = 8
    CE = CR * ROW
    nchunks = e // CE
    cb, crem = nchunks // (NC * NS), nchunks % (NC * NS)
    ch = (-(-n // NS) + 7) // 8 * 8
    last = n - (NS - 1) * ch

    mesh = plsc.VectorSubcoreMesh(
        core_axis_name="c", subcore_axis_name="s", num_cores=NC,
        num_subcores=NS)

    scratch = (
        [pltpu.VMEM_SHARED((n,), jnp.float32) for _ in range(D)]
        + [pltpu.VMEM_SHARED((n,), jnp.float32) for _ in range(npl)]
        + [pltpu.VMEM((ch,), jnp.float32),
           pltpu.VMEM((CE,), jnp.int32),
           pltpu.VMEM((CR, ROW), jnp.int32),
           pltpu.VMEM((CE,), jnp.float32)]
        + [pltpu.VMEM((CE,), jnp.float32) for _ in range(D)]
        + [pltpu.VMEM((ROW,), jnp.float32),
           pltpu.SemaphoreType.DMA,
           pltpu.SemaphoreType.DMA]
    )

    @functools.partial(
        pl.kernel,
        out_type=jax.ShapeDtypeStruct((NC * npl * n,), jnp.float32),
        mesh=mesh, scratch_types=scratch)
    def edge_pass(*a):
        h_in = a[0:D]
        src1, dst1, ew1, out = a[D], a[D + 1], a[D + 2], a[D + 3]
        sc = a[D + 4:]
        h_sh = sc[0:D]
        acc = sc[D:D + npl]
        zbuf, src_v, dst_v, ew_v = sc[D + npl:D + npl + 4]
        g = sc[D + npl + 4:D + npl + 4 + D]
        ones_v, sem_g, sem_s = sc[D + npl + 4 + D:D + npl + 4 + D + 3]

        cid = lax.axis_index("c")
        sid = lax.axis_index("s")
        off = sid * ch

        def ofill(i, _):
            ones_v[pl.ds(i * 16, 16)] = jnp.ones((16,), jnp.float32)
            return 0
        lax.fori_loop(0, ROW // 16, ofill, 0)

        @pl.when(sid < NS - 1)
        def _():
            for j in range(D):
                pltpu.sync_copy(h_in[j].at[pl.ds(off, ch)], zbuf)
                pltpu.sync_copy(zbuf, h_sh[j].at[pl.ds(off, ch)])

        @pl.when(sid == NS - 1)
        def _():
            for j in range(D):
                pltpu.sync_copy(h_in[j].at[pl.ds(off, last)],
                                zbuf.at[pl.ds(0, last)])
                pltpu.sync_copy(zbuf.at[pl.ds(0, last)],
                                h_sh[j].at[pl.ds(off, last)])

        def zfill(i, _):
            zbuf[pl.ds(i * 16, 16)] = jnp.zeros((16,), jnp.float32)
            return 0
        lax.fori_loop(0, ch // 16, zfill, 0)

        @pl.when(sid < NS - 1)
        def _():
            for j in range(npl):
                pltpu.sync_copy(zbuf, acc[j].at[pl.ds(off, ch)])

        @pl.when(sid == NS - 1)
        def _():
            for j in range(npl):
                pltpu.sync_copy(zbuf.at[pl.ds(0, last)],
                                acc[j].at[pl.ds(off, last)])

        plsc.subcore_barrier()

        wid = cid * NS + sid
        base = wid * cb + jnp.minimum(wid, crem)
        nch = cb + jnp.where(wid < crem, 1, 0)

        def chunk_body(t, _):
            eoff = (base + t) * CE
            roff = (base + t) * CR
            pltpu.sync_copy(src1.at[pl.ds(eoff, CE)], src_v)
            pltpu.sync_copy(ew1.at[pl.ds(eoff, CE)], ew_v)
            pltpu.sync_copy(dst1.at[pl.ds(roff, CR), :], dst_v)
            gd = []
            for j in range(D):
                gd.append(pltpu.async_copy(h_sh[j].at[src_v], g[j], sem_g))
            for dsc in gd:
                dsc.wait()

            def mul_body(k, _):
                sl = pl.ds(k * 16, 16)
                w = ew_v[sl]
                for j in range(D):
                    g[j][sl] = g[j][sl] * w
                return 0
            lax.fori_loop(0, CE // 16, mul_body, 0)

            sd = []
            for q in range(CR):
                for j in range(D):
                    sd.append(pltpu.async_copy(
                        g[j].at[pl.ds(q * ROW, ROW)],
                        acc[j].at[dst_v.at[q]], sem_s, add=True))
                if with_count:
                    sd.append(pltpu.async_copy(
                        ones_v, acc[D].at[dst_v.at[q]], sem_s, add=True))
            for dsc in sd:
                dsc.wait()
            return 0
        lax.fori_loop(0, nch, chunk_body, 0)

        plsc.subcore_barrier()

        @pl.when(sid < NS - 1)
        def _():
            for j in range(npl):
                pltpu.sync_copy(acc[j].at[pl.ds(off, ch)], zbuf)
                pltpu.sync_copy(zbuf,
                                out.at[pl.ds((cid * npl + j) * n + off, ch)])

        @pl.when(sid == NS - 1)
        def _():
            for j in range(npl):
                pltpu.sync_copy(acc[j].at[pl.ds(off, last)],
                                zbuf.at[pl.ds(0, last)])
                pltpu.sync_copy(zbuf.at[pl.ds(0, last)],
                                out.at[pl.ds((cid * npl + j) * n + off, last)])

    return edge_pass


def _dense_first(n):
    def body(part, wt, b, out, cnt_out):
        p = part[0] + part[1]
        c = jnp.maximum(p[D:D + 1, :], 1.0)
        m = p[0:D, :] / c
        x = jnp.dot(wt[...], m, preferred_element_type=jnp.float32) + b[...]
        out[...] = jnp.maximum(x, 0.0)
        cnt_out[...] = c

    return pl.pallas_call(
        body,
        out_shape=[jax.ShapeDtypeStruct((D, n), jnp.float32),
                   jax.ShapeDtypeStruct((1, n), jnp.float32)])


def _dense_mid(n):
    def body(part, cnt, wt, b, out):
        p = part[0] + part[1]
        m = p / cnt[...]
        x = jnp.dot(wt[...], m, preferred_element_type=jnp.float32) + b[...]
        out[...] = jnp.maximum(x, 0.0)

    return pl.pallas_call(
        body, out_shape=jax.ShapeDtypeStruct((D, n), jnp.float32))


def _dense_final(n):
    def body(part, cnt, wt3, b3, fwt1, fb1, fwt2, fb2, fwt3, fb3, out):
        p = part[0] + part[1]
        m = p / cnt[...]
        x = jnp.dot(wt3[...], m, preferred_element_type=jnp.float32) + b3[...]
        h = jnp.maximum(x, 0.0)
        x = jnp.dot(fwt1[...], h, preferred_element_type=jnp.float32) + fb1[...]
        h = jnp.maximum(x, 0.0)
        x = jnp.dot(fwt2[...], h, preferred_element_type=jnp.float32) + fb2[...]
        h = jnp.maximum(x, 0.0)
        out[...] = (jnp.dot(fwt3[...], h, preferred_element_type=jnp.float32)
                    + fb3[...])

    return pl.pallas_call(
        body, out_shape=jax.ShapeDtypeStruct((D, n), jnp.float32))


@functools.lru_cache(maxsize=4)
def _build(n, e):
    return (_edge_pass(n, e, True), _edge_pass(n, e, False),
            _dense_first(n), _dense_mid(n), _dense_final(n))


def kernel(h, edge_index, edge_weight, W1, b1, W2, b2, W3, b3,
           fcW1, fcb1, fcW2, fcb2, fcW3, fcb3):
    n, d = h.shape
    e = edge_weight.shape[0]
    assert d == D and e % ROW == 0
    ep1, ep, dfirst, dmid, dfinal = _build(n, e)

    src1 = edge_index[0]
    dst1 = edge_index[1].reshape(e // ROW, ROW)
    ew1 = edge_weight

    planes = [h[:, j] for j in range(D)]

    part = ep1(*planes, src1, dst1, ew1).reshape(NC, 6, n)
    h1, cnt = dfirst(part, W1.T, b1[:, None])

    part = ep(*[h1[j] for j in range(D)], src1, dst1, ew1).reshape(NC, D, n)
    h2 = dmid(part, cnt, W2.T, b2[:, None])

    part = ep(*[h2[j] for j in range(D)], src1, dst1, ew1).reshape(NC, D, n)
    y = dfinal(part, cnt, W3.T, b3[:, None], fcW1.T, fcb1[:, None],
               fcW2.T, fcb2[:, None], fcW3.T, fcb3[:, None])

    return y.T


# speedup vs baseline: 45.0867x; 1.2114x over previous
import functools

import jax
import jax.numpy as jnp
from jax import lax
from jax.experimental import pallas as pl
from jax.experimental.pallas import tpu as pltpu
from jax.experimental.pallas import tpu_sc as plsc

NC = 2
NS = 16
ROW = 128
D = 5


def _edge_pass(n, e, with_count):
    npl = 6 if with_count else 5
    CR = 16
    CE = CR * ROW
    nchunks = e // CE
    cb, crem = nchunks // (NC * NS), nchunks % (NC * NS)
    ch = (-(-n // NS) + 7) // 8 * 8
    last = n - (NS - 1) * ch

    mesh = plsc.VectorSubcoreMesh(
        core_axis_name="c", subcore_axis_name="s", num_cores=NC,
        num_subcores=NS)

    scratch = (
        [pltpu.VMEM_SHARED((n,), jnp.float32) for _ in range(D)]
        + [pltpu.VMEM_SHARED((n,), jnp.float32) for _ in range(npl)]
        + [pltpu.VMEM((ch,), jnp.float32),
           pltpu.VMEM((CE,), jnp.int32),
           pltpu.VMEM((CE,), jnp.int32),
           pltpu.VMEM((CE,), jnp.float32)]
        + [pltpu.VMEM((CE,), jnp.float32) for _ in range(D)]
        + [pltpu.VMEM((CE,), jnp.float32),
           pltpu.SemaphoreType.DMA,
           pltpu.SemaphoreType.DMA]
    )

    @functools.partial(
        pl.kernel,
        out_type=jax.ShapeDtypeStruct((NC * npl * n,), jnp.float32),
        mesh=mesh, scratch_types=scratch)
    def edge_pass(*a):
        h_in = a[0:D]
        src1, dst1, ew1, out = a[D], a[D + 1], a[D + 2], a[D + 3]
        sc = a[D + 4:]
        h_sh = sc[0:D]
        acc = sc[D:D + npl]
        zbuf, src_v, dst_v, ew_v = sc[D + npl:D + npl + 4]
        g = sc[D + npl + 4:D + npl + 4 + D]
        ones_v, sem_g, sem_s = sc[D + npl + 4 + D:D + npl + 4 + D + 3]

        cid = lax.axis_index("c")
        sid = lax.axis_index("s")
        off = sid * ch

        def ofill(i, _):
            ones_v[pl.ds(i * 16, 16)] = jnp.ones((16,), jnp.float32)
            return 0
        lax.fori_loop(0, CE // 16, ofill, 0)

        @pl.when(sid < NS - 1)
        def _():
            for j in range(D):
                pltpu.sync_copy(h_in[j].at[pl.ds(off, ch)], zbuf)
                pltpu.sync_copy(zbuf, h_sh[j].at[pl.ds(off, ch)])

        @pl.when(sid == NS - 1)
        def _():
            for j in range(D):
                pltpu.sync_copy(h_in[j].at[pl.ds(off, last)],
                                zbuf.at[pl.ds(0, last)])
                pltpu.sync_copy(zbuf.at[pl.ds(0, last)],
                                h_sh[j].at[pl.ds(off, last)])

        def zfill(i, _):
            zbuf[pl.ds(i * 16, 16)] = jnp.zeros((16,), jnp.float32)
            return 0
        lax.fori_loop(0, ch // 16, zfill, 0)

        @pl.when(sid < NS - 1)
        def _():
            for j in range(npl):
                pltpu.sync_copy(zbuf, acc[j].at[pl.ds(off, ch)])

        @pl.when(sid == NS - 1)
        def _():
            for j in range(npl):
                pltpu.sync_copy(zbuf.at[pl.ds(0, last)],
                                acc[j].at[pl.ds(off, last)])

        plsc.subcore_barrier()

        wid = cid * NS + sid
        base = wid * cb + jnp.minimum(wid, crem)
        nch = cb + jnp.where(wid < crem, 1, 0)

        def chunk_body(t, _):
            eoff = (base + t) * CE
            pltpu.sync_copy(src1.at[pl.ds(eoff, CE)], src_v)
            pltpu.sync_copy(ew1.at[pl.ds(eoff, CE)], ew_v)
            pltpu.sync_copy(dst1.at[pl.ds(eoff, CE)], dst_v)
            gd = []
            for j in range(D):
                gd.append(pltpu.async_copy(h_sh[j].at[src_v], g[j], sem_g))
            for dsc in gd:
                dsc.wait()

            def mul_body(k, _):
                sl = pl.ds(k * 16, 16)
                w = ew_v[sl]
                for j in range(D):
                    g[j][sl] = g[j][sl] * w
                return 0
            lax.fori_loop(0, CE // 16, mul_body, 0)

            sd = []
            for j in range(D):
                sd.append(pltpu.async_copy(
                    g[j], acc[j].at[dst_v], sem_s, add=True))
            if with_count:
                sd.append(pltpu.async_copy(
                    ones_v, acc[D].at[dst_v], sem_s, add=True))
            for dsc in sd:
                dsc.wait()
            return 0
        lax.fori_loop(0, nch, chunk_body, 0)

        plsc.subcore_barrier()

        @pl.when(sid < NS - 1)
        def _():
            for j in range(npl):
                pltpu.sync_copy(acc[j].at[pl.ds(off, ch)], zbuf)
                pltpu.sync_copy(zbuf,
                                out.at[pl.ds((cid * npl + j) * n + off, ch)])

        @pl.when(sid == NS - 1)
        def _():
            for j in range(npl):
                pltpu.sync_copy(acc[j].at[pl.ds(off, last)],
                                zbuf.at[pl.ds(0, last)])
                pltpu.sync_copy(zbuf.at[pl.ds(0, last)],
                                out.at[pl.ds((cid * npl + j) * n + off, last)])

    return edge_pass


def _dense_first(n):
    def body(part, wt, b, out, cnt_out):
        p = part[0] + part[1]
        c = jnp.maximum(p[D:D + 1, :], 1.0)
        m = p[0:D, :] / c
        x = jnp.dot(wt[...], m, preferred_element_type=jnp.float32) + b[...]
        out[...] = jnp.maximum(x, 0.0)
        cnt_out[...] = c

    return pl.pallas_call(
        body,
        out_shape=[jax.ShapeDtypeStruct((D, n), jnp.float32),
                   jax.ShapeDtypeStruct((1, n), jnp.float32)])


def _dense_mid(n):
    def body(part, cnt, wt, b, out):
        p = part[0] + part[1]
        m = p / cnt[...]
        x = jnp.dot(wt[...], m, preferred_element_type=jnp.float32) + b[...]
        out[...] = jnp.maximum(x, 0.0)

    return pl.pallas_call(
        body, out_shape=jax.ShapeDtypeStruct((D, n), jnp.float32))


def _dense_final(n):
    def body(part, cnt, wt3, b3, fwt1, fb1, fwt2, fb2, fwt3, fb3, out):
        p = part[0] + part[1]
        m = p / cnt[...]
        x = jnp.dot(wt3[...], m, preferred_element_type=jnp.float32) + b3[...]
        h = jnp.maximum(x, 0.0)
        x = jnp.dot(fwt1[...], h, preferred_element_type=jnp.float32) + fb1[...]
        h = jnp.maximum(x, 0.0)
        x = jnp.dot(fwt2[...], h, preferred_element_type=jnp.float32) + fb2[...]
        h = jnp.maximum(x, 0.0)
        out[...] = (jnp.dot(fwt3[...], h, preferred_element_type=jnp.float32)
                    + fb3[...])

    return pl.pallas_call(
        body, out_shape=jax.ShapeDtypeStruct((D, n), jnp.float32))


@functools.lru_cache(maxsize=4)
def _build(n, e):
    return (_edge_pass(n, e, True), _edge_pass(n, e, False),
            _dense_first(n), _dense_mid(n), _dense_final(n))


def kernel(h, edge_index, edge_weight, W1, b1, W2, b2, W3, b3,
           fcW1, fcb1, fcW2, fcb2, fcW3, fcb3):
    n, d = h.shape
    e = edge_weight.shape[0]
    assert d == D and e % ROW == 0
    ep1, ep, dfirst, dmid, dfinal = _build(n, e)

    src1 = edge_index[0]
    dst1 = edge_index[1]
    ew1 = edge_weight

    planes = [h[:, j] for j in range(D)]

    part = ep1(*planes, src1, dst1, ew1).reshape(NC, 6, n)
    h1, cnt = dfirst(part, W1.T, b1[:, None])

    part = ep(*[h1[j] for j in range(D)], src1, dst1, ew1).reshape(NC, D, n)
    h2 = dmid(part, cnt, W2.T, b2[:, None])

    part = ep(*[h2[j] for j in range(D)], src1, dst1, ew1).reshape(NC, D, n)
    y = dfinal(part, cnt, W3.T, b3[:, None], fcW1.T, fcb1[:, None],
               fcW2.T, fcb2[:, None], fcW3.T, fcb3[:, None])

    return y.T


# speedup vs baseline: 51.2720x; 1.1372x over previous
import functools

import jax
import jax.numpy as jnp
from jax import lax
from jax.experimental import pallas as pl
from jax.experimental.pallas import tpu as pltpu
from jax.experimental.pallas import tpu_sc as plsc

NC = 2
NS = 16
ROW = 128
D = 5


def _edge_pass(n, e, with_count):
    npl = 6 if with_count else 5
    CE = 3200
    nchunks = e // CE
    cb, crem = nchunks // (NC * NS), nchunks % (NC * NS)
    npairs = (cb + 2) // 2
    ch = (-(-n // NS) + 7) // 8 * 8
    last = n - (NS - 1) * ch

    mesh = plsc.VectorSubcoreMesh(
        core_axis_name="c", subcore_axis_name="s", num_cores=NC,
        num_subcores=NS)

    scratch = (
        [pltpu.VMEM_SHARED((n,), jnp.float32) for _ in range(D)]
        + [pltpu.VMEM_SHARED((n,), jnp.float32) for _ in range(npl)]
        + [pltpu.VMEM((ch,), jnp.float32)]
        + [pltpu.VMEM((CE,), jnp.int32)
           for _ in range(2)]
        + [pltpu.VMEM((CE,), jnp.int32) for _ in range(2)]
        + [pltpu.VMEM((CE,), jnp.float32) for _ in range(2)]
        + [pltpu.VMEM((CE,), jnp.float32) for _ in range(2 * D)]
        + [pltpu.VMEM((CE,), jnp.float32),
           pltpu.SemaphoreType.DMA,
           pltpu.SemaphoreType.DMA,
           pltpu.SemaphoreType.DMA,
           pltpu.SemaphoreType.DMA,
           pltpu.SemaphoreType.DMA]
    )

    @functools.partial(
        pl.kernel,
        out_type=jax.ShapeDtypeStruct((NC * npl * n,), jnp.float32),
        mesh=mesh, scratch_types=scratch)
    def edge_pass(*a):
        h_in = a[0:D]
        src1, dst1, ew1, out = a[D], a[D + 1], a[D + 2], a[D + 3]
        sc = a[D + 4:]
        h_sh = sc[0:D]
        acc = sc[D:D + npl]
        p = D + npl
        zbuf = sc[p]
        src_v = sc[p + 1:p + 3]
        dst_v = sc[p + 3:p + 5]
        ew_v = sc[p + 5:p + 7]
        g = [sc[p + 7:p + 7 + D], sc[p + 7 + D:p + 7 + 2 * D]]
        q = p + 7 + 2 * D
        ones_v, sem_g = sc[q], sc[q + 1]
        sem_l = sc[q + 2:q + 4]
        sem_s = sc[q + 4:q + 6]

        cid = lax.axis_index("c")
        sid = lax.axis_index("s")
        off = sid * ch

        def ofill(i, _):
            ones_v[pl.ds(i * 16, 16)] = jnp.ones((16,), jnp.float32)
            return 0
        lax.fori_loop(0, CE // 16, ofill, 0)

        @pl.when(sid < NS - 1)
        def _():
            for j in range(D):
                pltpu.sync_copy(h_in[j].at[pl.ds(off, ch)], zbuf)
                pltpu.sync_copy(zbuf, h_sh[j].at[pl.ds(off, ch)])

        @pl.when(sid == NS - 1)
        def _():
            for j in range(D):
                pltpu.sync_copy(h_in[j].at[pl.ds(off, last)],
                                zbuf.at[pl.ds(0, last)])
                pltpu.sync_copy(zbuf.at[pl.ds(0, last)],
                                h_sh[j].at[pl.ds(off, last)])

        def zfill(i, _):
            zbuf[pl.ds(i * 16, 16)] = jnp.zeros((16,), jnp.float32)
            return 0
        lax.fori_loop(0, ch // 16, zfill, 0)

        @pl.when(sid < NS - 1)
        def _():
            for j in range(npl):
                pltpu.sync_copy(zbuf, acc[j].at[pl.ds(off, ch)])

        @pl.when(sid == NS - 1)
        def _():
            for j in range(npl):
                pltpu.sync_copy(zbuf.at[pl.ds(0, last)],
                                acc[j].at[pl.ds(off, last)])

        plsc.subcore_barrier()

        wid = cid * NS + sid
        base = wid * cb + jnp.minimum(wid, crem)
        nch = cb + jnp.where(wid < crem, 1, 0)

        def issue_lin(s, t):
            eoff = (base + t) * CE
            pltpu.async_copy(src1.at[pl.ds(eoff, CE)], src_v[s], sem_l[s])
            pltpu.async_copy(ew1.at[pl.ds(eoff, CE)], ew_v[s], sem_l[s])
            pltpu.async_copy(dst1.at[pl.ds(eoff, CE)], dst_v[s], sem_l[s])

        def drain_lin(s):
            pltpu.make_async_copy(src1.at[pl.ds(0, CE)], src_v[s],
                                  sem_l[s]).wait()
            pltpu.make_async_copy(ew1.at[pl.ds(0, CE)], ew_v[s],
                                  sem_l[s]).wait()
            pltpu.make_async_copy(dst1.at[pl.ds(0, CE)], dst_v[s],
                                  sem_l[s]).wait()

        def drain_scat(s):
            for j in range(D):
                pltpu.make_async_copy(ew1.at[pl.ds(0, CE)], g[s][j],
                                      sem_s[s]).wait()
            if with_count:
                pltpu.make_async_copy(ew1.at[pl.ds(0, CE)], ones_v,
                                      sem_s[s]).wait()

        def half(s, t):
            @pl.when(t < nch)
            def _():
                drain_lin(s)
                @pl.when(t >= 2)
                def _():
                    drain_scat(s)
                gd = [pltpu.async_copy(h_sh[j].at[src_v[s]], g[s][j], sem_g)
                      for j in range(D)]
                @pl.when(t + 1 < nch)
                def _():
                    issue_lin(1 - s, t + 1)
                for dsc in gd:
                    dsc.wait()

                def mul_body(k, _):
                    sl = pl.ds(k * 16, 16)
                    w = ew_v[s][sl]
                    for j in range(D):
                        g[s][j][sl] = g[s][j][sl] * w
                    return 0
                lax.fori_loop(0, CE // 16, mul_body, 0)
                for j in range(D):
                    pltpu.async_copy(g[s][j], acc[j].at[dst_v[s]],
                                     sem_s[s], add=True)
                if with_count:
                    pltpu.async_copy(ones_v, acc[D].at[dst_v[s]],
                                     sem_s[s], add=True)

        issue_lin(0, 0)

        def pair_body(i, _):
            half(0, 2 * i)
            half(1, 2 * i + 1)
            return 0
        lax.fori_loop(0, npairs, pair_body, 0)

        drain_scat(0)
        @pl.when(nch >= 2)
        def _():
            drain_scat(1)

        plsc.subcore_barrier()

        @pl.when(sid < NS - 1)
        def _():
            for j in range(npl):
                pltpu.sync_copy(acc[j].at[pl.ds(off, ch)], zbuf)
                pltpu.sync_copy(zbuf,
                                out.at[pl.ds((cid * npl + j) * n + off, ch)])

        @pl.when(sid == NS - 1)
        def _():
            for j in range(npl):
                pltpu.sync_copy(acc[j].at[pl.ds(off, last)],
                                zbuf.at[pl.ds(0, last)])
                pltpu.sync_copy(zbuf.at[pl.ds(0, last)],
                                out.at[pl.ds((cid * npl + j) * n + off, last)])

    return edge_pass


def _dense_first(n):
    def body(part, wt, b, out, cnt_out):
        p = part[0] + part[1]
        c = jnp.maximum(p[D:D + 1, :], 1.0)
        m = p[0:D, :] / c
        x = jnp.dot(wt[...], m, preferred_element_type=jnp.float32) + b[...]
        out[...] = jnp.maximum(x, 0.0)
        cnt_out[...] = c

    return pl.pallas_call(
        body,
        out_shape=[jax.ShapeDtypeStruct((D, n), jnp.float32),
                   jax.ShapeDtypeStruct((1, n), jnp.float32)])


def _dense_mid(n):
    def body(part, cnt, wt, b, out):
        p = part[0] + part[1]
        m = p / cnt[...]
        x = jnp.dot(wt[...], m, preferred_element_type=jnp.float32) + b[...]
        out[...] = jnp.maximum(x, 0.0)

    return pl.pallas_call(
        body, out_shape=jax.ShapeDtypeStruct((D, n), jnp.float32))


def _dense_final(n):
    def body(part, cnt, wt3, b3, fwt1, fb1, fwt2, fb2, fwt3, fb3, out):
        p = part[0] + part[1]
        m = p / cnt[...]
        x = jnp.dot(wt3[...], m, preferred_element_type=jnp.float32) + b3[...]
        h = jnp.maximum(x, 0.0)
        x = jnp.dot(fwt1[...], h, preferred_element_type=jnp.float32) + fb1[...]
        h = jnp.maximum(x, 0.0)
        x = jnp.dot(fwt2[...], h, preferred_element_type=jnp.float32) + fb2[...]
        h = jnp.maximum(x, 0.0)
        out[...] = (jnp.dot(fwt3[...], h, preferred_element_type=jnp.float32)
                    + fb3[...])

    return pl.pallas_call(
        body, out_shape=jax.ShapeDtypeStruct((D, n), jnp.float32))


@functools.lru_cache(maxsize=4)
def _build(n, e):
    return (_edge_pass(n, e, True), _edge_pass(n, e, False),
            _dense_first(n), _dense_mid(n), _dense_final(n))


def kernel(h, edge_index, edge_weight, W1, b1, W2, b2, W3, b3,
           fcW1, fcb1, fcW2, fcb2, fcW3, fcb3):
    n, d = h.shape
    e = edge_weight.shape[0]
    assert d == D and e % ROW == 0
    ep1, ep, dfirst, dmid, dfinal = _build(n, e)

    src1 = edge_index[0]
    dst1 = edge_index[1]
    ew1 = edge_weight

    planes = [h[:, j] for j in range(D)]

    part = ep1(*planes, src1, dst1, ew1).reshape(NC, 6, n)
    h1, cnt = dfirst(part, W1.T, b1[:, None])

    part = ep(*[h1[j] for j in range(D)], src1, dst1, ew1).reshape(NC, D, n)
    h2 = dmid(part, cnt, W2.T, b2[:, None])

    part = ep(*[h2[j] for j in range(D)], src1, dst1, ew1).reshape(NC, D, n)
    y = dfinal(part, cnt, W3.T, b3[:, None], fcW1.T, fcb1[:, None],
               fcW2.T, fcb2[:, None], fcW3.T, fcb3[:, None])

    return y.T
